# initial kernel scaffold (unmeasured)
import jax
import jax.numpy as jnp
from jax import lax
from jax.experimental import pallas as pl
from jax.experimental.pallas import tpu as pltpu

N_DEV = 32
N_LAYERS = 3


def kernel(x, Win0, Wout0, Win1, Wout1, Win2, Wout2):
    b, d = x.shape
    rows_per = b // N_DEV

    def body(x_ref, win0_ref, wout0_ref, win1_ref, wout1_ref, win2_ref,
             wout2_ref, out_ref, comm_ref, send_buf, send_sem, recv_sems):
        my = lax.axis_index("i")

        barrier = pltpu.get_barrier_semaphore()
        for off in range(1, N_DEV):
            pl.semaphore_signal(
                barrier, inc=1,
                device_id=((my + off) % N_DEV,),
                device_id_type=pl.DeviceIdType.MESH,
            )
        pl.semaphore_wait(barrier, N_DEV - 1)

        wins = [win0_ref, win1_ref, win2_ref]
        wouts = [wout0_ref, wout1_ref, wout2_ref]

        xv = x_ref[:, :].astype(jnp.bfloat16)
        total = None
        for l in range(N_LAYERS):
            h = jnp.dot(xv, wins[l][:, :].astype(jnp.bfloat16),
                        preferred_element_type=jnp.float32)
            h = jnp.maximum(h, 0.0).astype(jnp.bfloat16)
            partial = jnp.dot(h, wouts[l][:, :].astype(jnp.bfloat16),
                              preferred_element_type=jnp.float32)

            send_buf[l, :, :] = partial.astype(jnp.bfloat16)

            rdmas = []
            for off in range(1, N_DEV):
                rdma = pltpu.make_async_remote_copy(
                    src_ref=send_buf.at[l],
                    dst_ref=comm_ref.at[l, off - 1],
                    send_sem=send_sem,
                    recv_sem=recv_sems.at[l],
                    device_id=((my + off) % N_DEV,),
                    device_id_type=pl.DeviceIdType.MESH,
                )
                rdma.start()
                rdmas.append(rdma)
            for r in rdmas:
                r.wait_recv()
            others = jnp.sum(comm_ref[l].astype(jnp.float32), axis=0)
            total = partial + others
            xv = total.astype(jnp.bfloat16)
            for r in rdmas:
                r.wait_send()

        out_ref[:, :] = lax.dynamic_slice(
            total, (my * rows_per, 0), (rows_per, d))

    return pl.pallas_call(
        body,
        out_shape=jax.ShapeDtypeStruct((rows_per, d), jnp.float32),
        in_specs=[pl.BlockSpec(memory_space=pltpu.VMEM)] * 7,
        out_specs=pl.BlockSpec(memory_space=pltpu.VMEM),
        scratch_shapes=[
            pltpu.VMEM((N_LAYERS, N_DEV - 1, b, d), jnp.bfloat16),
            pltpu.VMEM((N_LAYERS, b, d), jnp.bfloat16),
            pltpu.SemaphoreType.DMA,
            pltpu.SemaphoreType.DMA((N_LAYERS,)),
        ],
        compiler_params=pltpu.CompilerParams(collective_id=0),
    )(x, Win0, Wout0, Win1, Wout1, Win2, Wout2)


# baseline (device time: 56457 ns/iter reference)
import jax
import jax.numpy as jnp
from jax import lax
from jax.experimental import pallas as pl
from jax.experimental.pallas import tpu as pltpu

N_DEV = 32
N_LAYERS = 3


def kernel(x, Win0, Wout0, Win1, Wout1, Win2, Wout2):
    b, d = x.shape
    rows_per = b // N_DEV

    def body(x_ref, win0_ref, wout0_ref, win1_ref, wout1_ref, win2_ref,
             wout2_ref, out_ref, comm_ref, send_buf, total_buf, send_sem,
             recv_sems):
        my = lax.axis_index("i")

        barrier = pltpu.get_barrier_semaphore()
        for off in range(1, N_DEV):
            pl.semaphore_signal(
                barrier, inc=1,
                device_id=((my + off) % N_DEV,),
                device_id_type=pl.DeviceIdType.MESH,
            )
        pl.semaphore_wait(barrier, N_DEV - 1)

        wins = [win0_ref, win1_ref, win2_ref]
        wouts = [wout0_ref, wout1_ref, wout2_ref]

        xv = x_ref[:, :].astype(jnp.bfloat16)
        total = None
        for l in range(N_LAYERS):
            h = jnp.dot(xv, wins[l][:, :].astype(jnp.bfloat16),
                        preferred_element_type=jnp.float32)
            h = jnp.maximum(h, 0.0).astype(jnp.bfloat16)
            partial = jnp.dot(h, wouts[l][:, :].astype(jnp.bfloat16),
                              preferred_element_type=jnp.float32)

            send_buf[l, :, :] = partial.astype(jnp.bfloat16)

            rdmas = []
            for off in range(1, N_DEV):
                rdma = pltpu.make_async_remote_copy(
                    src_ref=send_buf.at[l],
                    dst_ref=comm_ref.at[l, off - 1],
                    send_sem=send_sem,
                    recv_sem=recv_sems.at[l],
                    device_id=((my + off) % N_DEV,),
                    device_id_type=pl.DeviceIdType.MESH,
                )
                rdma.start()
                rdmas.append(rdma)
            for r in rdmas:
                r.wait_recv()
            others = jnp.sum(comm_ref[l].astype(jnp.float32), axis=0)
            total = partial + others
            xv = total.astype(jnp.bfloat16)
            for r in rdmas:
                r.wait_send()

        total_buf[:, :] = total
        out_ref[:, :] = total_buf[pl.ds(my * rows_per, rows_per), :]

    return pl.pallas_call(
        body,
        out_shape=jax.ShapeDtypeStruct((rows_per, d), jnp.float32),
        in_specs=[pl.BlockSpec(memory_space=pltpu.VMEM)] * 7,
        out_specs=pl.BlockSpec(memory_space=pltpu.VMEM),
        scratch_shapes=[
            pltpu.VMEM((N_LAYERS, N_DEV - 1, b, d), jnp.bfloat16),
            pltpu.VMEM((N_LAYERS, b, d), jnp.bfloat16),
            pltpu.VMEM((b, d), jnp.float32),
            pltpu.SemaphoreType.DMA,
            pltpu.SemaphoreType.DMA((N_LAYERS,)),
        ],
        compiler_params=pltpu.CompilerParams(collective_id=0),
    )(x, Win0, Wout0, Win1, Wout1, Win2, Wout2)


# device time: 38800 ns/iter; 1.4551x vs baseline; 1.4551x over previous
import jax
import jax.numpy as jnp
from jax import lax
from jax.experimental import pallas as pl
from jax.experimental.pallas import tpu as pltpu

N_DEV = 32
N_LAYERS = 3
G = 4
N_GROUPS = N_DEV // G


def kernel(x, Win0, Wout0, Win1, Wout1, Win2, Wout2):
    b, d = x.shape
    rows_per = b // N_DEV

    def body(x_ref, win0_ref, wout0_ref, win1_ref, wout1_ref, win2_ref,
             wout2_ref, out_ref, comm1_ref, comm2_ref, send1_buf, send2_buf,
             total_buf, send_sem, recv1_sems, recv2_sems):
        my = lax.axis_index("i")
        grp = (my // G) * G
        k = my % G

        barrier = pltpu.get_barrier_semaphore()
        for j in range(1, G):
            pl.semaphore_signal(
                barrier, inc=1,
                device_id=(grp + (k + j) % G,),
                device_id_type=pl.DeviceIdType.MESH,
            )
        for j in range(1, N_GROUPS):
            pl.semaphore_signal(
                barrier, inc=1,
                device_id=((grp + j * G) % N_DEV + k,),
                device_id_type=pl.DeviceIdType.MESH,
            )
        pl.semaphore_wait(barrier, (G - 1) + (N_GROUPS - 1))

        wins = [win0_ref, win1_ref, win2_ref]
        wouts = [wout0_ref, wout1_ref, wout2_ref]

        xv = x_ref[:, :].astype(jnp.bfloat16)
        total = None
        for l in range(N_LAYERS):
            h = jnp.dot(xv, wins[l][:, :].astype(jnp.bfloat16),
                        preferred_element_type=jnp.float32)
            h = jnp.maximum(h, 0.0).astype(jnp.bfloat16)
            partial = jnp.dot(h, wouts[l][:, :].astype(jnp.bfloat16),
                              preferred_element_type=jnp.float32)

            send1_buf[l, :, :] = partial.astype(jnp.bfloat16)
            p1 = []
            for j in range(1, G):
                rdma = pltpu.make_async_remote_copy(
                    src_ref=send1_buf.at[l],
                    dst_ref=comm1_ref.at[l, j - 1],
                    send_sem=send_sem,
                    recv_sem=recv1_sems.at[l],
                    device_id=(grp + (k + j) % G,),
                    device_id_type=pl.DeviceIdType.MESH,
                )
                rdma.start()
                p1.append(rdma)
            for r in p1:
                r.wait_recv()
            gsum = partial + jnp.sum(comm1_ref[l].astype(jnp.float32), axis=0)

            send2_buf[l, :, :] = gsum.astype(jnp.bfloat16)
            p2 = []
            for j in range(1, N_GROUPS):
                rdma = pltpu.make_async_remote_copy(
                    src_ref=send2_buf.at[l],
                    dst_ref=comm2_ref.at[l, j - 1],
                    send_sem=send_sem,
                    recv_sem=recv2_sems.at[l],
                    device_id=((grp + j * G) % N_DEV + k,),
                    device_id_type=pl.DeviceIdType.MESH,
                )
                rdma.start()
                p2.append(rdma)
            for r in p2:
                r.wait_recv()
            total = gsum + jnp.sum(comm2_ref[l].astype(jnp.float32), axis=0)
            xv = total.astype(jnp.bfloat16)
            for r in p1:
                r.wait_send()
            for r in p2:
                r.wait_send()

        total_buf[:, :] = total
        out_ref[:, :] = total_buf[pl.ds(my * rows_per, rows_per), :]

    return pl.pallas_call(
        body,
        out_shape=jax.ShapeDtypeStruct((rows_per, d), jnp.float32),
        in_specs=[pl.BlockSpec(memory_space=pltpu.VMEM)] * 7,
        out_specs=pl.BlockSpec(memory_space=pltpu.VMEM),
        scratch_shapes=[
            pltpu.VMEM((N_LAYERS, G - 1, b, d), jnp.bfloat16),
            pltpu.VMEM((N_LAYERS, N_GROUPS - 1, b, d), jnp.bfloat16),
            pltpu.VMEM((N_LAYERS, b, d), jnp.bfloat16),
            pltpu.VMEM((N_LAYERS, b, d), jnp.bfloat16),
            pltpu.VMEM((b, d), jnp.float32),
            pltpu.SemaphoreType.DMA,
            pltpu.SemaphoreType.DMA((N_LAYERS,)),
            pltpu.SemaphoreType.DMA((N_LAYERS,)),
        ],
        compiler_params=pltpu.CompilerParams(collective_id=0),
    )(x, Win0, Wout0, Win1, Wout1, Win2, Wout2)


# device time: 35371 ns/iter; 1.5961x vs baseline; 1.0969x over previous
import jax
import jax.numpy as jnp
from jax import lax
from jax.experimental import pallas as pl
from jax.experimental.pallas import tpu as pltpu

N_DEV = 32
N_LAYERS = 3
G = 4
N_GROUPS = N_DEV // G


def kernel(x, Win0, Wout0, Win1, Wout1, Win2, Wout2):
    b, d = x.shape
    rows_per = b // N_DEV

    def body(x_ref, win0_ref, wout0_ref, win1_ref, wout1_ref, win2_ref,
             wout2_ref, out_ref, comm1_ref, comm2_ref, send1_buf, send2_buf,
             comm_rs_ref, send_rs_buf, send_sem, recv1_sems, recv2_sems,
             recv_rs_sem):
        my = lax.axis_index("i")
        grp = (my // G) * G
        k = my % G

        barrier = pltpu.get_barrier_semaphore()
        for j in range(1, G):
            pl.semaphore_signal(
                barrier, inc=1,
                device_id=(grp + (k + j) % G,),
                device_id_type=pl.DeviceIdType.MESH,
            )
        for j in range(1, N_GROUPS):
            pl.semaphore_signal(
                barrier, inc=1,
                device_id=((grp + j * G) % N_DEV + k,),
                device_id_type=pl.DeviceIdType.MESH,
            )
        pl.semaphore_wait(barrier, (G - 1) + (N_GROUPS - 1))

        wins = [win0_ref, win1_ref, win2_ref]
        wouts = [wout0_ref, wout1_ref, wout2_ref]

        xv = x_ref[:, :].astype(jnp.bfloat16)
        for l in range(N_LAYERS - 1):
            h = jnp.dot(xv, wins[l][:, :].astype(jnp.bfloat16),
                        preferred_element_type=jnp.float32)
            h = jnp.maximum(h, 0.0).astype(jnp.bfloat16)
            partial = jnp.dot(h, wouts[l][:, :].astype(jnp.bfloat16),
                              preferred_element_type=jnp.float32)

            send1_buf[l, :, :] = partial.astype(jnp.bfloat16)
            p1 = []
            for j in range(1, G):
                rdma = pltpu.make_async_remote_copy(
                    src_ref=send1_buf.at[l],
                    dst_ref=comm1_ref.at[l, j - 1],
                    send_sem=send_sem,
                    recv_sem=recv1_sems.at[l],
                    device_id=(grp + (k + j) % G,),
                    device_id_type=pl.DeviceIdType.MESH,
                )
                rdma.start()
                p1.append(rdma)
            for r in p1:
                r.wait_recv()
            gsum = partial + jnp.sum(comm1_ref[l].astype(jnp.float32), axis=0)

            send2_buf[l, :, :] = gsum.astype(jnp.bfloat16)
            p2 = []
            for j in range(1, N_GROUPS):
                rdma = pltpu.make_async_remote_copy(
                    src_ref=send2_buf.at[l],
                    dst_ref=comm2_ref.at[l, j - 1],
                    send_sem=send_sem,
                    recv_sem=recv2_sems.at[l],
                    device_id=((grp + j * G) % N_DEV + k,),
                    device_id_type=pl.DeviceIdType.MESH,
                )
                rdma.start()
                p2.append(rdma)
            for r in p2:
                r.wait_recv()
            total = gsum + jnp.sum(comm2_ref[l].astype(jnp.float32), axis=0)
            xv = total.astype(jnp.bfloat16)
            for r in p1:
                r.wait_send()
            for r in p2:
                r.wait_send()

        h = jnp.dot(xv, wins[2][:, :].astype(jnp.bfloat16),
                    preferred_element_type=jnp.float32)
        h = jnp.maximum(h, 0.0).astype(jnp.bfloat16)
        partial = jnp.dot(h, wouts[2][:, :].astype(jnp.bfloat16),
                          preferred_element_type=jnp.float32)
        send_rs_buf[:, :] = partial
        rs = []
        for off in range(1, N_DEV):
            tgt = (my + off) % N_DEV
            rdma = pltpu.make_async_remote_copy(
                src_ref=send_rs_buf.at[pl.ds(tgt * rows_per, rows_per), :],
                dst_ref=comm_rs_ref.at[off - 1],
                send_sem=send_sem,
                recv_sem=recv_rs_sem,
                device_id=(tgt,),
                device_id_type=pl.DeviceIdType.MESH,
            )
            rdma.start()
            rs.append(rdma)
        for r in rs:
            r.wait_recv()
        own = send_rs_buf[pl.ds(my * rows_per, rows_per), :]
        out_ref[:, :] = own + jnp.sum(comm_rs_ref[:, :, :], axis=0)
        for r in rs:
            r.wait_send()

    return pl.pallas_call(
        body,
        out_shape=jax.ShapeDtypeStruct((rows_per, d), jnp.float32),
        in_specs=[pl.BlockSpec(memory_space=pltpu.VMEM)] * 7,
        out_specs=pl.BlockSpec(memory_space=pltpu.VMEM),
        scratch_shapes=[
            pltpu.VMEM((N_LAYERS, G - 1, b, d), jnp.bfloat16),
            pltpu.VMEM((N_LAYERS, N_GROUPS - 1, b, d), jnp.bfloat16),
            pltpu.VMEM((N_LAYERS, b, d), jnp.bfloat16),
            pltpu.VMEM((N_LAYERS, b, d), jnp.bfloat16),
            pltpu.VMEM((N_DEV - 1, rows_per, d), jnp.float32),
            pltpu.VMEM((b, d), jnp.float32),
            pltpu.SemaphoreType.DMA,
            pltpu.SemaphoreType.DMA((N_LAYERS,)),
            pltpu.SemaphoreType.DMA((N_LAYERS,)),
            pltpu.SemaphoreType.DMA,
        ],
        compiler_params=pltpu.CompilerParams(collective_id=0),
    )(x, Win0, Wout0, Win1, Wout1, Win2, Wout2)


# device time: 34960 ns/iter; 1.6149x vs baseline; 1.0118x over previous
import jax
import jax.numpy as jnp
from jax import lax
from jax.experimental import pallas as pl
from jax.experimental.pallas import tpu as pltpu

N_DEV = 32
N_LAYERS = 3
G = 4
N_GROUPS = N_DEV // G


def kernel(x, Win0, Wout0, Win1, Wout1, Win2, Wout2):
    b, d = x.shape
    rows_per = b // N_DEV

    def body(x_ref, win0_ref, wout0_ref, win1_ref, wout1_ref, win2_ref,
             wout2_ref, out_ref, comm1_ref, comm2_ref, send1_buf, send2_buf,
             comm_rs_ref, send_rs_buf, send_sem, recv1_sems, recv2_sems,
             recv_rs_sem):
        my = lax.axis_index("i")
        grp = (my // G) * G
        k = my % G

        barrier = pltpu.get_barrier_semaphore()
        for j in range(1, G):
            pl.semaphore_signal(
                barrier, inc=1,
                device_id=(grp + (k + j) % G,),
                device_id_type=pl.DeviceIdType.MESH,
            )
        for j in range(1, N_GROUPS):
            pl.semaphore_signal(
                barrier, inc=1,
                device_id=((grp + j * G) % N_DEV + k,),
                device_id_type=pl.DeviceIdType.MESH,
            )

        wins = [win0_ref, win1_ref, win2_ref]
        wouts = [wout0_ref, wout1_ref, wout2_ref]

        pending_sends = []
        xv = x_ref[:, :].astype(jnp.bfloat16)
        for l in range(N_LAYERS - 1):
            h = jnp.dot(xv, wins[l][:, :].astype(jnp.bfloat16),
                        preferred_element_type=jnp.float32)
            h = jnp.maximum(h, 0.0).astype(jnp.bfloat16)
            partial = jnp.dot(h, wouts[l][:, :].astype(jnp.bfloat16),
                              preferred_element_type=jnp.float32)

            send1_buf[l, :, :] = partial.astype(jnp.bfloat16)
            if l == 0:
                pl.semaphore_wait(barrier, (G - 1) + (N_GROUPS - 1))
            p1 = []
            for j in range(1, G):
                rdma = pltpu.make_async_remote_copy(
                    src_ref=send1_buf.at[l],
                    dst_ref=comm1_ref.at[l, j - 1],
                    send_sem=send_sem,
                    recv_sem=recv1_sems.at[l],
                    device_id=(grp + (k + j) % G,),
                    device_id_type=pl.DeviceIdType.MESH,
                )
                rdma.start()
                p1.append(rdma)
            for r in p1:
                r.wait_recv()
            gsum = partial + jnp.sum(comm1_ref[l].astype(jnp.float32), axis=0)

            send2_buf[l, :, :] = gsum.astype(jnp.bfloat16)
            p2 = []
            for j in range(1, N_GROUPS):
                rdma = pltpu.make_async_remote_copy(
                    src_ref=send2_buf.at[l],
                    dst_ref=comm2_ref.at[l, j - 1],
                    send_sem=send_sem,
                    recv_sem=recv2_sems.at[l],
                    device_id=((grp + j * G) % N_DEV + k,),
                    device_id_type=pl.DeviceIdType.MESH,
                )
                rdma.start()
                p2.append(rdma)
            for r in p2:
                r.wait_recv()
            total = gsum + jnp.sum(comm2_ref[l].astype(jnp.float32), axis=0)
            xv = total.astype(jnp.bfloat16)
            pending_sends += p1 + p2

        h = jnp.dot(xv, wins[2][:, :].astype(jnp.bfloat16),
                    preferred_element_type=jnp.float32)
        h = jnp.maximum(h, 0.0).astype(jnp.bfloat16)
        partial = jnp.dot(h, wouts[2][:, :].astype(jnp.bfloat16),
                          preferred_element_type=jnp.float32)
        send_rs_buf[:, :] = partial
        rs = []
        for off in range(1, N_DEV):
            tgt = (my + off) % N_DEV
            rdma = pltpu.make_async_remote_copy(
                src_ref=send_rs_buf.at[pl.ds(tgt * rows_per, rows_per), :],
                dst_ref=comm_rs_ref.at[off - 1],
                send_sem=send_sem,
                recv_sem=recv_rs_sem,
                device_id=(tgt,),
                device_id_type=pl.DeviceIdType.MESH,
            )
            rdma.start()
            rs.append(rdma)
        for r in rs:
            r.wait_recv()
        own = send_rs_buf[pl.ds(my * rows_per, rows_per), :]
        out_ref[:, :] = own + jnp.sum(comm_rs_ref[:, :, :], axis=0)
        for r in pending_sends + rs:
            r.wait_send()

    return pl.pallas_call(
        body,
        out_shape=jax.ShapeDtypeStruct((rows_per, d), jnp.float32),
        in_specs=[pl.BlockSpec(memory_space=pltpu.VMEM)] * 7,
        out_specs=pl.BlockSpec(memory_space=pltpu.VMEM),
        scratch_shapes=[
            pltpu.VMEM((N_LAYERS, G - 1, b, d), jnp.bfloat16),
            pltpu.VMEM((N_LAYERS, N_GROUPS - 1, b, d), jnp.bfloat16),
            pltpu.VMEM((N_LAYERS, b, d), jnp.bfloat16),
            pltpu.VMEM((N_LAYERS, b, d), jnp.bfloat16),
            pltpu.VMEM((N_DEV - 1, rows_per, d), jnp.float32),
            pltpu.VMEM((b, d), jnp.float32),
            pltpu.SemaphoreType.DMA,
            pltpu.SemaphoreType.DMA((N_LAYERS,)),
            pltpu.SemaphoreType.DMA((N_LAYERS,)),
            pltpu.SemaphoreType.DMA,
        ],
        compiler_params=pltpu.CompilerParams(collective_id=0),
    )(x, Win0, Wout0, Win1, Wout1, Win2, Wout2)
